# Initial kernel scaffold; baseline (speedup 1.0000x reference)
import functools

import jax
import jax.numpy as jnp
from jax import lax
from jax.experimental import pallas as pl
from jax.experimental.pallas import tpu as pltpu
from jax.experimental.pallas import tpu_sc as plsc

NCORE = 2
NSUB = 16
NW = NCORE * NSUB
K = 128
TRASH = 240
ROW_BLK = 1000
DEG_W = 16


def _sc_mesh():
    return plsc.VectorSubcoreMesh(core_axis_name="c", subcore_axis_name="s")


def _make_deg(C, n_acc):
    rpt = n_acc // NSUB

    @functools.partial(
        pl.kernel,
        out_type=jax.ShapeDtypeStruct((NCORE, n_acc, DEG_W), jnp.float32),
        mesh=_sc_mesh(),
        scratch_types=[
            pltpu.VMEM_SHARED((n_acc, DEG_W), jnp.float32),
            pltpu.VMEM((K,), jnp.int32),
            pltpu.VMEM((K, DEG_W), jnp.float32),
            pltpu.VMEM((K, DEG_W), jnp.float32),
        ],
    )
    def deg_k(dsti_hbm, ones_hbm, zeros_hbm, out_hbm, acc, di, ones_v, zeros_v):
        cid = lax.axis_index("c")
        sid = lax.axis_index("s")
        pltpu.sync_copy(zeros_hbm, zeros_v)
        pltpu.sync_copy(ones_hbm, ones_v)

        @pl.loop(0, rpt, step=K)
        def _(r):
            pltpu.sync_copy(zeros_v, acc.at[pl.ds(sid * rpt + r, K)])

        plsc.subcore_barrier()

        @pl.loop(0, C)
        def _(c):
            pltpu.sync_copy(dsti_hbm.at[cid, sid, c], di)
            pltpu.sync_copy(ones_v, acc.at[di], add=True)

        plsc.subcore_barrier()
        pltpu.sync_copy(acc.at[pl.ds(sid * rpt, rpt)],
                        out_hbm.at[cid, pl.ds(sid * rpt, rpt)])

    return deg_k


def _make_prop(F, C, n_acc):
    rpt = n_acc // NSUB

    @functools.partial(
        pl.kernel,
        out_type=jax.ShapeDtypeStruct((NCORE, n_acc, F), jnp.float32),
        mesh=_sc_mesh(),
        scratch_types=[
            pltpu.VMEM_SHARED((n_acc, F), jnp.float32),
            pltpu.VMEM((K,), jnp.int32),
            pltpu.VMEM((K,), jnp.int32),
            pltpu.VMEM((K, F), jnp.float32),
            pltpu.VMEM((K, F), jnp.float32),
            pltpu.SemaphoreType.DMA,
        ],
    )
    def prop_k(y_hbm, srci_hbm, dsti_hbm, zeros_hbm, out_hbm,
               acc, si, di, rows, zeros_v, sem):
        cid = lax.axis_index("c")
        sid = lax.axis_index("s")
        pltpu.sync_copy(zeros_hbm, zeros_v)

        @pl.loop(0, rpt, step=K)
        def _(r):
            pltpu.sync_copy(zeros_v, acc.at[pl.ds(sid * rpt + r, K)])

        plsc.subcore_barrier()

        @pl.loop(0, C)
        def _(c):
            pltpu.sync_copy(srci_hbm.at[cid, sid, c], si)
            pltpu.sync_copy(dsti_hbm.at[cid, sid, c], di)
            pltpu.async_copy(y_hbm.at[si], rows, sem).wait()
            pltpu.sync_copy(rows, acc.at[di], add=True)

        plsc.subcore_barrier()
        pltpu.sync_copy(acc.at[pl.ds(sid * rpt, rpt)],
                        out_hbm.at[cid, pl.ds(sid * rpt, rpt)])

    return prop_k


def _mm_body(x_ref, w_ref, o_ref):
    o_ref[...] = jnp.dot(x_ref[...], w_ref[...],
                         preferred_element_type=jnp.float32)


def _y1_body(degp_ref, xw_ref, y_ref, dinv_ref):
    deg = degp_ref[0, :, 0:1] + degp_ref[1, :, 0:1] + 1.0
    dinv = lax.rsqrt(deg)
    dinv_ref[...] = dinv
    y_ref[...] = dinv * xw_ref[...]


def _mid_body(p_ref, y_ref, dinv_ref, w_ref, o_ref, *, leaky):
    dinv = dinv_ref[...]
    t = dinv * (p_ref[0] + p_ref[1] + y_ref[...])
    if leaky:
        t = jnp.where(t >= 0, t, 0.3 * t)
    else:
        t = jnp.maximum(t, 0.0)
    o_ref[...] = dinv * jnp.dot(t, w_ref[...],
                                preferred_element_type=jnp.float32)


def _fin_body(p_ref, y_ref, dinv_ref, bmv_ref, gi_ref, o_ref, pool_ref, acc_ref):
    i = pl.program_id(0)
    dinv = dinv_ref[...]
    out = dinv * (p_ref[0] + p_ref[1] + y_ref[...]) + bmv_ref[...]
    o_ref[...] = out

    @pl.when(i == 0)
    def _():
        acc_ref[...] = jnp.zeros_like(acc_ref)

    onehot = (gi_ref[...] == lax.broadcasted_iota(
        jnp.int32, (ROW_BLK, 64), 1)).astype(jnp.float32)
    m33 = jnp.concatenate(
        [out[:, :32], jnp.ones((ROW_BLK, 1), jnp.float32)], axis=1)
    acc_ref[...] += lax.dot_general(
        onehot, m33, (((0,), (0,)), ((), ())),
        preferred_element_type=jnp.float32)

    @pl.when(i == pl.num_programs(0) - 1)
    def _():
        a = acc_ref[...]
        pool_ref[...] = a[:, :32] / jnp.maximum(a[:, 32:33], 1.0)


def kernel(x, edge_index, graph_index, W1, W2, Wm, bm, Wv, bv):
    n, d = x.shape
    e = edge_index.shape[1]
    h1 = W1.shape[1]
    h2 = W2.shape[1]
    z2 = Wm.shape[1] + Wv.shape[1]
    src, dst = edge_index[0], edge_index[1]

    C = -(-e // (NW * K))
    pad = NW * C * K - e
    n_acc = n + TRASH
    assert n_acc % (NSUB * K) == 0
    ar = jnp.arange(pad, dtype=jnp.int32)
    src_p = jnp.concatenate([src, (ar * 131) % n])
    dst_p = jnp.concatenate([dst, n + (ar % TRASH)])
    srci = src_p.reshape(NCORE, NSUB, C, K)
    dsti = dst_p.reshape(NCORE, NSUB, C, K)

    zeros_deg = jnp.zeros((K, DEG_W), jnp.float32)
    ones_deg = jnp.ones((K, DEG_W), jnp.float32)
    zeros_h1 = jnp.zeros((K, h1), jnp.float32)
    zeros_h2 = jnp.zeros((K, h2), jnp.float32)

    grid = (n // ROW_BLK,)

    degp = _make_deg(C, n_acc)(dsti, ones_deg, zeros_deg)

    xw1 = pl.pallas_call(
        _mm_body,
        grid=grid,
        in_specs=[pl.BlockSpec((ROW_BLK, d), lambda i: (i, 0)),
                  pl.BlockSpec((d, h1), lambda i: (0, 0))],
        out_specs=pl.BlockSpec((ROW_BLK, h1), lambda i: (i, 0)),
        out_shape=jax.ShapeDtypeStruct((n, h1), jnp.float32),
    )(x, W1)

    y1, dinv = pl.pallas_call(
        _y1_body,
        grid=grid,
        in_specs=[pl.BlockSpec((NCORE, ROW_BLK, DEG_W), lambda i: (0, i, 0)),
                  pl.BlockSpec((ROW_BLK, h1), lambda i: (i, 0))],
        out_specs=[pl.BlockSpec((ROW_BLK, h1), lambda i: (i, 0)),
                   pl.BlockSpec((ROW_BLK, 1), lambda i: (i, 0))],
        out_shape=[jax.ShapeDtypeStruct((n, h1), jnp.float32),
                   jax.ShapeDtypeStruct((n, 1), jnp.float32)],
    )(degp, xw1)

    p1 = _make_prop(h1, C, n_acc)(y1, srci, dsti, zeros_h1)

    y2 = pl.pallas_call(
        functools.partial(_mid_body, leaky=True),
        grid=grid,
        in_specs=[pl.BlockSpec((NCORE, ROW_BLK, h1), lambda i: (0, i, 0)),
                  pl.BlockSpec((ROW_BLK, h1), lambda i: (i, 0)),
                  pl.BlockSpec((ROW_BLK, 1), lambda i: (i, 0)),
                  pl.BlockSpec((h1, h2), lambda i: (0, 0))],
        out_specs=pl.BlockSpec((ROW_BLK, h2), lambda i: (i, 0)),
        out_shape=jax.ShapeDtypeStruct((n, h2), jnp.float32),
    )(p1, y1, dinv, W2)

    p2 = _make_prop(h2, C, n_acc)(y2, srci, dsti, zeros_h2)

    Wmv = jnp.concatenate([Wm, Wv], axis=1)
    y3 = pl.pallas_call(
        functools.partial(_mid_body, leaky=False),
        grid=grid,
        in_specs=[pl.BlockSpec((NCORE, ROW_BLK, h2), lambda i: (0, i, 0)),
                  pl.BlockSpec((ROW_BLK, h2), lambda i: (i, 0)),
                  pl.BlockSpec((ROW_BLK, 1), lambda i: (i, 0)),
                  pl.BlockSpec((h2, z2), lambda i: (0, 0))],
        out_specs=pl.BlockSpec((ROW_BLK, z2), lambda i: (i, 0)),
        out_shape=jax.ShapeDtypeStruct((n, z2), jnp.float32),
    )(p2, y2, dinv, Wmv)

    p3 = _make_prop(z2, C, n_acc)(y3, srci, dsti, zeros_h2)

    bmv = jnp.concatenate([bm, bv]).reshape(1, z2)
    gi = graph_index.reshape(n, 1)
    out, pool = pl.pallas_call(
        _fin_body,
        grid=grid,
        in_specs=[pl.BlockSpec((NCORE, ROW_BLK, z2), lambda i: (0, i, 0)),
                  pl.BlockSpec((ROW_BLK, z2), lambda i: (i, 0)),
                  pl.BlockSpec((ROW_BLK, 1), lambda i: (i, 0)),
                  pl.BlockSpec((1, z2), lambda i: (0, 0)),
                  pl.BlockSpec((ROW_BLK, 1), lambda i: (i, 0))],
        out_specs=[pl.BlockSpec((ROW_BLK, z2), lambda i: (i, 0)),
                   pl.BlockSpec((64, 32), lambda i: (0, 0))],
        out_shape=[jax.ShapeDtypeStruct((n, z2), jnp.float32),
                   jax.ShapeDtypeStruct((64, 32), jnp.float32)],
        scratch_shapes=[pltpu.VMEM((64, 33), jnp.float32)],
    )(p3, y3, dinv, bmv, gi)

    return out[:, :32], out[:, 32:], pool


# speedup vs baseline: 15.3942x; 15.3942x over previous
import functools

import jax
import jax.numpy as jnp
from jax import lax
from jax.experimental import pallas as pl
from jax.experimental.pallas import tpu as pltpu
from jax.experimental.pallas import tpu_sc as plsc

NCORE = 2
NSUB = 16
NW = NCORE * NSUB
K = 128
TRASH = 240
ROW_BLK = 1000
DEG_W = 16


def _sc_mesh():
    return plsc.VectorSubcoreMesh(core_axis_name="c", subcore_axis_name="s")


def _make_deg(C, n_acc):
    rpt = n_acc // NSUB

    @functools.partial(
        pl.kernel,
        out_type=jax.ShapeDtypeStruct((NCORE, n_acc, DEG_W), jnp.float32),
        mesh=_sc_mesh(),
        scratch_types=[
            pltpu.VMEM_SHARED((n_acc, DEG_W), jnp.float32),
            pltpu.VMEM((K,), jnp.int32),
            pltpu.VMEM((K, DEG_W), jnp.float32),
            pltpu.VMEM((K, DEG_W), jnp.float32),
        ],
    )
    def deg_k(dsti_hbm, ones_hbm, zeros_hbm, out_hbm, acc, di, ones_v, zeros_v):
        cid = lax.axis_index("c")
        sid = lax.axis_index("s")
        pltpu.sync_copy(zeros_hbm, zeros_v)
        pltpu.sync_copy(ones_hbm, ones_v)

        @pl.loop(0, rpt, step=K)
        def _(r):
            pltpu.sync_copy(zeros_v, acc.at[pl.ds(sid * rpt + r, K)])

        plsc.subcore_barrier()

        @pl.loop(0, C)
        def _(c):
            pltpu.sync_copy(dsti_hbm.at[cid, sid, c], di)
            pltpu.sync_copy(ones_v, acc.at[di], add=True)

        plsc.subcore_barrier()
        pltpu.sync_copy(acc.at[pl.ds(sid * rpt, rpt)],
                        out_hbm.at[cid, pl.ds(sid * rpt, rpt)])

    return deg_k


def _make_prop(F, C, n_acc):
    rpt = n_acc // NSUB

    @functools.partial(
        pl.kernel,
        out_type=jax.ShapeDtypeStruct((NCORE, n_acc, F), jnp.float32),
        mesh=_sc_mesh(),
        scratch_types=[
            pltpu.VMEM_SHARED((n_acc, F), jnp.float32),
            pltpu.VMEM((K,), jnp.int32),
            pltpu.VMEM((K,), jnp.int32),
            pltpu.VMEM((K, F), jnp.float32),
            pltpu.VMEM((K, F), jnp.float32),
            pltpu.SemaphoreType.DMA,
        ],
    )
    def prop_k(y_hbm, srci_hbm, dsti_hbm, zeros_hbm, out_hbm,
               acc, si, di, rows, zeros_v, sem):
        cid = lax.axis_index("c")
        sid = lax.axis_index("s")
        pltpu.sync_copy(zeros_hbm, zeros_v)

        @pl.loop(0, rpt, step=K)
        def _(r):
            pltpu.sync_copy(zeros_v, acc.at[pl.ds(sid * rpt + r, K)])

        plsc.subcore_barrier()

        @pl.loop(0, C)
        def _(c):
            pltpu.sync_copy(srci_hbm.at[cid, sid, c], si)
            pltpu.sync_copy(dsti_hbm.at[cid, sid, c], di)
            pltpu.async_copy(y_hbm.at[si], rows, sem).wait()
            pltpu.sync_copy(rows, acc.at[di], add=True)

        plsc.subcore_barrier()
        pltpu.sync_copy(acc.at[pl.ds(sid * rpt, rpt)],
                        out_hbm.at[cid, pl.ds(sid * rpt, rpt)])

    return prop_k


def _mm_body(x_ref, w_ref, o_ref):
    o_ref[...] = jnp.dot(x_ref[...], w_ref[...],
                         preferred_element_type=jnp.float32)


def _y1_body(degp_ref, xw_ref, y_ref, dinv_ref):
    deg = degp_ref[0, :, 0:1] + degp_ref[1, :, 0:1] + 1.0
    dinv = lax.rsqrt(deg)
    dinv_ref[...] = dinv
    y_ref[...] = dinv * xw_ref[...]


def _mid_body(p_ref, y_ref, dinv_ref, w_ref, o_ref, *, leaky):
    dinv = dinv_ref[...]
    w_in = w_ref.shape[0]
    t = dinv * (p_ref[0][:, :w_in] + p_ref[1][:, :w_in] +
                y_ref[...][:, :w_in])
    if leaky:
        t = jnp.where(t >= 0, t, 0.3 * t)
    else:
        t = jnp.maximum(t, 0.0)
    res = dinv * jnp.dot(t, w_ref[...], preferred_element_type=jnp.float32)
    pad = o_ref.shape[1] - res.shape[1]
    if pad:
        res = jnp.concatenate(
            [res, jnp.zeros((res.shape[0], pad), jnp.float32)], axis=1)
    o_ref[...] = res


def _fin_body(p_ref, y_ref, dinv_ref, bmv_ref, gi_ref, o_ref, pool_ref, acc_ref):
    i = pl.program_id(0)
    dinv = dinv_ref[...]
    w = bmv_ref.shape[1]
    out = dinv * (p_ref[0][:, :w] + p_ref[1][:, :w] +
                  y_ref[...][:, :w]) + bmv_ref[...]
    o_ref[...] = out

    @pl.when(i == 0)
    def _():
        acc_ref[...] = jnp.zeros_like(acc_ref)

    onehot = (gi_ref[...] == lax.broadcasted_iota(
        jnp.int32, (ROW_BLK, 64), 1)).astype(jnp.float32)
    m33 = jnp.concatenate(
        [out[:, :32], jnp.ones((ROW_BLK, 1), jnp.float32)], axis=1)
    acc_ref[...] += lax.dot_general(
        onehot, m33, (((0,), (0,)), ((), ())),
        preferred_element_type=jnp.float32)

    @pl.when(i == pl.num_programs(0) - 1)
    def _():
        a = acc_ref[...]
        pool_ref[...] = a[:, :32] / jnp.maximum(a[:, 32:33], 1.0)


def kernel(x, edge_index, graph_index, W1, W2, Wm, bm, Wv, bv):
    n, d = x.shape
    e = edge_index.shape[1]
    h1 = W1.shape[1]
    h2 = W2.shape[1]
    z2 = Wm.shape[1] + Wv.shape[1]
    src, dst = edge_index[0], edge_index[1]

    C = -(-e // (NW * K))
    pad = NW * C * K - e
    n_acc = n + TRASH
    assert n_acc % (NSUB * K) == 0
    ar = jnp.arange(pad, dtype=jnp.int32)
    src_p = jnp.concatenate([src, (ar * 131) % n])
    dst_p = jnp.concatenate([dst, n + (ar % TRASH)])
    srci = src_p.reshape(NCORE, NSUB, C, K)
    dsti = dst_p.reshape(NCORE, NSUB, C, K)

    zeros_deg = jnp.zeros((K, DEG_W), jnp.float32)
    ones_deg = jnp.ones((K, DEG_W), jnp.float32)
    zeros_h1 = jnp.zeros((K, h1), jnp.float32)

    grid = (n // ROW_BLK,)

    degp = _make_deg(C, n_acc)(dsti, ones_deg, zeros_deg)

    xw1 = pl.pallas_call(
        _mm_body,
        grid=grid,
        in_specs=[pl.BlockSpec((ROW_BLK, d), lambda i: (i, 0)),
                  pl.BlockSpec((d, h1), lambda i: (0, 0))],
        out_specs=pl.BlockSpec((ROW_BLK, h1), lambda i: (i, 0)),
        out_shape=jax.ShapeDtypeStruct((n, h1), jnp.float32),
    )(x, W1)

    y1, dinv = pl.pallas_call(
        _y1_body,
        grid=grid,
        in_specs=[pl.BlockSpec((NCORE, ROW_BLK, DEG_W), lambda i: (0, i, 0)),
                  pl.BlockSpec((ROW_BLK, h1), lambda i: (i, 0))],
        out_specs=[pl.BlockSpec((ROW_BLK, h1), lambda i: (i, 0)),
                   pl.BlockSpec((ROW_BLK, 1), lambda i: (i, 0))],
        out_shape=[jax.ShapeDtypeStruct((n, h1), jnp.float32),
                   jax.ShapeDtypeStruct((n, 1), jnp.float32)],
    )(degp, xw1)

    p1 = _make_prop(h1, C, n_acc)(y1, srci, dsti, zeros_h1)

    y2 = pl.pallas_call(
        functools.partial(_mid_body, leaky=True),
        grid=grid,
        in_specs=[pl.BlockSpec((NCORE, ROW_BLK, h1), lambda i: (0, i, 0)),
                  pl.BlockSpec((ROW_BLK, h1), lambda i: (i, 0)),
                  pl.BlockSpec((ROW_BLK, 1), lambda i: (i, 0)),
                  pl.BlockSpec((h1, h2), lambda i: (0, 0))],
        out_specs=pl.BlockSpec((ROW_BLK, h1), lambda i: (i, 0)),
        out_shape=jax.ShapeDtypeStruct((n, h1), jnp.float32),
    )(p1, y1, dinv, W2)

    p2 = _make_prop(h1, C, n_acc)(y2, srci, dsti, zeros_h1)

    Wmv = jnp.concatenate([Wm, Wv], axis=1)
    y3 = pl.pallas_call(
        functools.partial(_mid_body, leaky=False),
        grid=grid,
        in_specs=[pl.BlockSpec((NCORE, ROW_BLK, h1), lambda i: (0, i, 0)),
                  pl.BlockSpec((ROW_BLK, h1), lambda i: (i, 0)),
                  pl.BlockSpec((ROW_BLK, 1), lambda i: (i, 0)),
                  pl.BlockSpec((h2, z2), lambda i: (0, 0))],
        out_specs=pl.BlockSpec((ROW_BLK, h1), lambda i: (i, 0)),
        out_shape=jax.ShapeDtypeStruct((n, h1), jnp.float32),
    )(p2, y2, dinv, Wmv)

    p3 = _make_prop(h1, C, n_acc)(y3, srci, dsti, zeros_h1)

    bmv = jnp.concatenate([bm, bv]).reshape(1, z2)
    gi = graph_index.reshape(n, 1)
    out, pool = pl.pallas_call(
        _fin_body,
        grid=grid,
        in_specs=[pl.BlockSpec((NCORE, ROW_BLK, h1), lambda i: (0, i, 0)),
                  pl.BlockSpec((ROW_BLK, h1), lambda i: (i, 0)),
                  pl.BlockSpec((ROW_BLK, 1), lambda i: (i, 0)),
                  pl.BlockSpec((1, z2), lambda i: (0, 0)),
                  pl.BlockSpec((ROW_BLK, 1), lambda i: (i, 0))],
        out_specs=[pl.BlockSpec((ROW_BLK, z2), lambda i: (i, 0)),
                   pl.BlockSpec((64, 32), lambda i: (0, 0))],
        out_shape=[jax.ShapeDtypeStruct((n, z2), jnp.float32),
                   jax.ShapeDtypeStruct((64, 32), jnp.float32)],
        scratch_shapes=[pltpu.VMEM((64, 33), jnp.float32)],
    )(p3, y3, dinv, bmv, gi)

    return out[:, :32], out[:, 32:], pool


# speedup vs baseline: 16.6107x; 1.0790x over previous
import functools

import jax
import jax.numpy as jnp
from jax import lax
from jax.experimental import pallas as pl
from jax.experimental.pallas import tpu as pltpu
from jax.experimental.pallas import tpu_sc as plsc

NCORE = 2
NSUB = 16
NW = NCORE * NSUB
K = 128
TRASH = 240
ROW_BLK = 1000
DEG_W = 16


def _sc_mesh():
    return plsc.VectorSubcoreMesh(core_axis_name="c", subcore_axis_name="s")


def _make_deg(C, n_acc):
    rpt = n_acc // NSUB

    @functools.partial(
        pl.kernel,
        out_type=jax.ShapeDtypeStruct((NCORE, n_acc, DEG_W), jnp.float32),
        mesh=_sc_mesh(),
        scratch_types=[
            pltpu.VMEM_SHARED((n_acc, DEG_W), jnp.float32),
            pltpu.VMEM((C, K), jnp.int32),
            pltpu.VMEM((K, DEG_W), jnp.float32),
            pltpu.VMEM((K, DEG_W), jnp.float32),
        ],
    )
    def deg_k(dsti_hbm, ones_hbm, zeros_hbm, out_hbm, acc, div, ones_v,
              zeros_v):
        cid = lax.axis_index("c")
        sid = lax.axis_index("s")

        pltpu.sync_copy(dsti_hbm.at[cid, sid], div)
        pltpu.sync_copy(zeros_hbm, zeros_v)
        pltpu.sync_copy(ones_hbm, ones_v)

        @pl.loop(0, rpt, step=K)
        def _(r):
            pltpu.sync_copy(zeros_v, acc.at[pl.ds(sid * rpt + r, K)])

        plsc.subcore_barrier()

        @pl.loop(0, C)
        def _(c):
            pltpu.sync_copy(ones_v, acc.at[div.at[c]], add=True)

        plsc.subcore_barrier()
        pltpu.sync_copy(acc.at[pl.ds(sid * rpt, rpt)],
                        out_hbm.at[cid, pl.ds(sid * rpt, rpt)])

    return deg_k


def _make_prop(F, C, n_acc, n_stage=0):
    rpt = n_acc // NSUB

    stage_scratch = [pltpu.VMEM_SHARED((n_stage, F), jnp.float32)] \
        if n_stage else []

    @functools.partial(
        pl.kernel,
        out_type=jax.ShapeDtypeStruct((NCORE, n_acc, F), jnp.float32),
        mesh=_sc_mesh(),
        scratch_types=[
            pltpu.VMEM_SHARED((n_acc, F), jnp.float32),
            *stage_scratch,
            pltpu.VMEM((C, K), jnp.int32),
            pltpu.VMEM((C, K), jnp.int32),
            pltpu.VMEM((K, F), jnp.float32),
            pltpu.SemaphoreType.DMA,
        ],
    )
    def prop_k(y_hbm, srci_hbm, dsti_hbm, zeros_hbm, out_hbm,
               acc, *rest):
        if n_stage:
            ystage, siv, div, rows, sem = rest
        else:
            siv, div, rows, sem = rest
        cid = lax.axis_index("c")
        sid = lax.axis_index("s")

        pltpu.sync_copy(srci_hbm.at[cid, sid], siv)
        pltpu.sync_copy(dsti_hbm.at[cid, sid], div)

        if n_stage:
            full = rpt * (NSUB - 1)

            @pl.when(sid < NSUB - 1)
            def _():
                pltpu.sync_copy(y_hbm.at[pl.ds(sid * rpt, rpt)],
                                ystage.at[pl.ds(sid * rpt, rpt)])

            @pl.when(sid == NSUB - 1)
            def _():
                pltpu.sync_copy(y_hbm.at[pl.ds(full, n_stage - full)],
                                ystage.at[pl.ds(full, n_stage - full)])

            ysrc = ystage
        else:
            ysrc = y_hbm

        pltpu.sync_copy(zeros_hbm, rows)

        @pl.loop(0, rpt, step=K)
        def _(r):
            pltpu.sync_copy(rows, acc.at[pl.ds(sid * rpt + r, K)])

        plsc.subcore_barrier()

        @pl.loop(0, C)
        def _(c):
            pltpu.async_copy(ysrc.at[siv.at[c]], rows, sem).wait()
            pltpu.sync_copy(rows, acc.at[div.at[c]], add=True)

        plsc.subcore_barrier()
        pltpu.sync_copy(acc.at[pl.ds(sid * rpt, rpt)],
                        out_hbm.at[cid, pl.ds(sid * rpt, rpt)])

    return prop_k


def _mm_body(x_ref, w_ref, o_ref):
    o_ref[...] = jnp.dot(x_ref[...], w_ref[...],
                         preferred_element_type=jnp.float32)


def _y1_body(degp_ref, xw_ref, y_ref, dinv_ref):
    deg = degp_ref[0, :, 0:1] + degp_ref[1, :, 0:1] + 1.0
    dinv = lax.rsqrt(deg)
    dinv_ref[...] = dinv
    y_ref[...] = dinv * xw_ref[...]


def _mid_body(p_ref, y_ref, dinv_ref, w_ref, o_ref, *, leaky):
    dinv = dinv_ref[...]
    w_in = w_ref.shape[0]
    t = dinv * (p_ref[0][:, :w_in] + p_ref[1][:, :w_in] +
                y_ref[...][:, :w_in])
    if leaky:
        t = jnp.where(t >= 0, t, 0.3 * t)
    else:
        t = jnp.maximum(t, 0.0)
    res = dinv * jnp.dot(t, w_ref[...], preferred_element_type=jnp.float32)
    pad = o_ref.shape[1] - res.shape[1]
    if pad:
        res = jnp.concatenate(
            [res, jnp.zeros((res.shape[0], pad), jnp.float32)], axis=1)
    o_ref[...] = res


def _fin_body(p_ref, y_ref, dinv_ref, bmv_ref, gi_ref, o_ref, pool_ref, acc_ref):
    i = pl.program_id(0)
    dinv = dinv_ref[...]
    w = bmv_ref.shape[1]
    out = dinv * (p_ref[0][:, :w] + p_ref[1][:, :w] +
                  y_ref[...][:, :w]) + bmv_ref[...]
    o_ref[...] = out

    @pl.when(i == 0)
    def _():
        acc_ref[...] = jnp.zeros_like(acc_ref)

    onehot = (gi_ref[...] == lax.broadcasted_iota(
        jnp.int32, (ROW_BLK, 64), 1)).astype(jnp.float32)
    m33 = jnp.concatenate(
        [out[:, :32], jnp.ones((ROW_BLK, 1), jnp.float32)], axis=1)
    acc_ref[...] += lax.dot_general(
        onehot, m33, (((0,), (0,)), ((), ())),
        preferred_element_type=jnp.float32)

    @pl.when(i == pl.num_programs(0) - 1)
    def _():
        a = acc_ref[...]
        pool_ref[...] = a[:, :32] / jnp.maximum(a[:, 32:33], 1.0)


def kernel(x, edge_index, graph_index, W1, W2, Wm, bm, Wv, bv):
    n, d = x.shape
    e = edge_index.shape[1]
    h1 = W1.shape[1]
    h2 = W2.shape[1]
    z2 = Wm.shape[1] + Wv.shape[1]
    src, dst = edge_index[0], edge_index[1]

    C = -(-e // (NW * K))
    C = -(-C // 2) * 2
    pad = NW * C * K - e
    n_acc = n + TRASH
    assert n_acc % (NSUB * K) == 0
    ar = jnp.arange(pad, dtype=jnp.int32)
    src_p = jnp.concatenate([src, (ar * 131) % n])
    dst_p = jnp.concatenate([dst, n + (ar % TRASH)])
    srci = src_p.reshape(NCORE, NSUB, C, K)
    dsti = dst_p.reshape(NCORE, NSUB, C, K)

    zeros_deg = jnp.zeros((K, DEG_W), jnp.float32)
    ones_deg = jnp.ones((K, DEG_W), jnp.float32)
    zeros_h1 = jnp.zeros((K, h1), jnp.float32)
    zeros_h2 = jnp.zeros((K, h2), jnp.float32)

    grid = (n // ROW_BLK,)

    degp = _make_deg(C, n_acc)(dsti, ones_deg, zeros_deg)

    xw1 = pl.pallas_call(
        _mm_body,
        grid=grid,
        in_specs=[pl.BlockSpec((ROW_BLK, d), lambda i: (i, 0)),
                  pl.BlockSpec((d, h1), lambda i: (0, 0))],
        out_specs=pl.BlockSpec((ROW_BLK, h1), lambda i: (i, 0)),
        out_shape=jax.ShapeDtypeStruct((n, h1), jnp.float32),
    )(x, W1)

    y1, dinv = pl.pallas_call(
        _y1_body,
        grid=grid,
        in_specs=[pl.BlockSpec((NCORE, ROW_BLK, DEG_W), lambda i: (0, i, 0)),
                  pl.BlockSpec((ROW_BLK, h1), lambda i: (i, 0))],
        out_specs=[pl.BlockSpec((ROW_BLK, h1), lambda i: (i, 0)),
                   pl.BlockSpec((ROW_BLK, 1), lambda i: (i, 0))],
        out_shape=[jax.ShapeDtypeStruct((n, h1), jnp.float32),
                   jax.ShapeDtypeStruct((n, 1), jnp.float32)],
    )(degp, xw1)

    p1 = _make_prop(h1, C, n_acc)(y1, srci, dsti, zeros_h1)

    y2 = pl.pallas_call(
        functools.partial(_mid_body, leaky=True),
        grid=grid,
        in_specs=[pl.BlockSpec((NCORE, ROW_BLK, h1), lambda i: (0, i, 0)),
                  pl.BlockSpec((ROW_BLK, h1), lambda i: (i, 0)),
                  pl.BlockSpec((ROW_BLK, 1), lambda i: (i, 0)),
                  pl.BlockSpec((h1, h2), lambda i: (0, 0))],
        out_specs=pl.BlockSpec((ROW_BLK, h1), lambda i: (i, 0)),
        out_shape=jax.ShapeDtypeStruct((n, h1), jnp.float32),
    )(p1, y1, dinv, W2)

    p2 = _make_prop(h1, C, n_acc)(y2, srci, dsti, zeros_h1)

    Wmv = jnp.concatenate([Wm, Wv], axis=1)
    y3 = pl.pallas_call(
        functools.partial(_mid_body, leaky=False),
        grid=grid,
        in_specs=[pl.BlockSpec((NCORE, ROW_BLK, h1), lambda i: (0, i, 0)),
                  pl.BlockSpec((ROW_BLK, h1), lambda i: (i, 0)),
                  pl.BlockSpec((ROW_BLK, 1), lambda i: (i, 0)),
                  pl.BlockSpec((h2, z2), lambda i: (0, 0))],
        out_specs=pl.BlockSpec((ROW_BLK, h1), lambda i: (i, 0)),
        out_shape=jax.ShapeDtypeStruct((n, h1), jnp.float32),
    )(p2, y2, dinv, Wmv)

    p3 = _make_prop(h1, C, n_acc)(y3, srci, dsti, zeros_h1)

    bmv = jnp.concatenate([bm, bv]).reshape(1, z2)
    gi = graph_index.reshape(n, 1)
    out, pool = pl.pallas_call(
        _fin_body,
        grid=grid,
        in_specs=[pl.BlockSpec((NCORE, ROW_BLK, h1), lambda i: (0, i, 0)),
                  pl.BlockSpec((ROW_BLK, h1), lambda i: (i, 0)),
                  pl.BlockSpec((ROW_BLK, 1), lambda i: (i, 0)),
                  pl.BlockSpec((1, z2), lambda i: (0, 0)),
                  pl.BlockSpec((ROW_BLK, 1), lambda i: (i, 0))],
        out_specs=[pl.BlockSpec((ROW_BLK, z2), lambda i: (i, 0)),
                   pl.BlockSpec((64, 32), lambda i: (0, 0))],
        out_shape=[jax.ShapeDtypeStruct((n, z2), jnp.float32),
                   jax.ShapeDtypeStruct((64, 32), jnp.float32)],
        scratch_shapes=[pltpu.VMEM((64, 33), jnp.float32)],
    )(p3, y3, dinv, bmv, gi)

    return out[:, :32], out[:, 32:], pool
